# baseline (speedup 1.0000x reference)
import functools

import jax
import jax.numpy as jnp
from jax import lax
from jax.experimental import pallas as pl
from jax.experimental.pallas import tpu as pltpu
from jax.experimental.pallas import tpu_sc as plsc

_N = 10000
_E = 320000
_D = 128
_H = 32
_NC = 2
_NS = 16
_NW = _NC * _NS
_EPW = _E // _NW
_KB = 80
_NBB = _EPW // _KB
_NP = 10240
_RPT = _NP // _NS
_ZR = 40
_EB = 2048
_E2 = 327680
_RB = 1000
_NG = _N // _RB


def _sc_dsq_body(pos_x, pos_y, pos_z, src, dst, dsq, px, py, pz, sv, dv, ov):
    c = lax.axis_index("c")
    s = lax.axis_index("s")
    wid = c * _NS + s
    base = wid * _EPW
    pltpu.sync_copy(pos_x, px)
    pltpu.sync_copy(pos_y, py)
    pltpu.sync_copy(pos_z, pz)
    pltpu.sync_copy(src.at[pl.ds(base, _EPW)], sv)
    pltpu.sync_copy(dst.at[pl.ds(base, _EPW)], dv)

    def body(i, carry):
        sl = pl.ds(i * 16, 16)
        si = sv[sl]
        di = dv[sl]
        dx = plsc.load_gather(px, [si]) - plsc.load_gather(px, [di])
        dy = plsc.load_gather(py, [si]) - plsc.load_gather(py, [di])
        dz = plsc.load_gather(pz, [si]) - plsc.load_gather(pz, [di])
        ov[sl] = dx * dx + dy * dy + dz * dz
        return carry

    lax.fori_loop(0, _EPW // 16, body, 0)
    pltpu.sync_copy(ov, dsq.at[pl.ds(base, _EPW)])


def _sc_conv_body(g, rw, src, dst, out, acc,
                  rows0, rows1, rwv0, rwv1,
                  si0, si1, si2, si3, di0, di1, di2, di3,
                  gs0, gs1, rs0, rs1, ss0, ss1,
                  is0, is1, is2, is3):
    c = lax.axis_index("c")
    s = lax.axis_index("s")
    wid = c * _NS + s

    rowsb = (rows0, rows1)
    rwvb = (rwv0, rwv1)
    sib = (si0, si1, si2, si3)
    dib = (di0, di1, di2, di3)
    gsem = (gs0, gs1)
    rsem = (rs0, rs1)
    ssem = (ss0, ss1)
    isem = (is0, is1, is2, is3)

    ebase = wid * _EPW
    rwbase = ebase * (_D // 2)

    def zfill(i, carry):
        for j in range(_D // 16):
            z = jnp.zeros((16,), jnp.float32)
            rows0[i, pl.ds(j * 16, 16)] = z
            rows1[i, pl.ds(j * 16, 16)] = z
        return carry

    lax.fori_loop(0, _KB, zfill, 0)
    for r in range(_RPT // _ZR):
        pltpu.sync_copy(rows0.at[pl.ds(0, _ZR), :],
                        acc.at[pl.ds(s * _RPT + r * _ZR, _ZR), :])
    plsc.subcore_barrier()

    def issue_idx(b, r):
        pltpu.async_copy(src.at[pl.ds(ebase + b * _KB, _KB)], sib[r],
                         isem[r])
        pltpu.async_copy(dst.at[pl.ds(ebase + b * _KB, _KB)], dib[r],
                         isem[r])

    def wait_idx(r):
        pltpu.make_async_copy(src.at[pl.ds(0, _KB)], sib[r],
                              isem[r]).wait()
        pltpu.make_async_copy(dst.at[pl.ds(0, _KB)], dib[r],
                              isem[r]).wait()

    _KW = _KB * (_D // 2)

    def issue_gr(b, q, r):
        pltpu.async_copy(g.at[sib[r]], rowsb[q], gsem[q])
        pltpu.async_copy(rw.at[pl.ds(rwbase + b * _KW, _KW)],
                         rwvb[q], rsem[q])

    def wait_gr(q, r):
        pltpu.make_async_copy(g.at[sib[r]], rowsb[q], gsem[q]).wait()
        pltpu.make_async_copy(rw.at[pl.ds(0, _KW)], rwvb[q],
                              rsem[q]).wait()

    def scat(q, r):
        pltpu.async_copy(rowsb[q], acc.at[dib[r]], ssem[q], add=True)

    def wait_s(q):
        pltpu.make_async_copy(rowsb[q], acc.at[dib[0]], ssem[q]).wait()

    def mul(q):
        rows = rowsb[q]
        rwv = rwvb[q]

        @plsc.parallel_loop(0, _KB, unroll=2)
        def _(e):
            for j in range(_D // 32):
                w = rwv[pl.ds(e * (_D // 2) + j * 16, 16)]
                lo = plsc.bitcast(w << 16, jnp.float32)
                hi = plsc.bitcast(w & jnp.int32(-65536), jnp.float32)
                sl0 = pl.ds(j * 32, 16)
                sl1 = pl.ds(j * 32 + 16, 16)
                rows[e, sl0] = rows[e, sl0] * lo
                rows[e, sl1] = rows[e, sl1] * hi

    issue_idx(0, 0)
    issue_idx(1, 1)
    wait_idx(0)
    pltpu.async_copy(rows1, acc.at[dib[0]], ssem[1], add=True)
    issue_gr(0, 0, 0)

    def step(b, q, r):
        qo = 1 - q
        rn1 = (r + 1) % 4
        rn2 = (r + 2) % 4
        wait_s(qo)
        issue_idx(jnp.minimum(b + 2, _NBB - 1), rn2)
        wait_idx(rn1)
        issue_gr(b + 1, qo, rn1)
        wait_gr(q, r)
        mul(q)
        scat(q, r)

    def quad(t, carry):
        b0 = 4 * t
        step(b0, 0, 0)
        step(b0 + 1, 1, 1)
        step(b0 + 2, 0, 2)
        step(b0 + 3, 1, 3)
        return carry

    lax.fori_loop(0, (_NBB - 1) // 4, quad, 0)
    wait_gr(0, 0)
    mul(0)
    scat(0, 0)
    wait_s(1)
    wait_s(0)
    wait_idx(1)
    plsc.subcore_barrier()
    pltpu.sync_copy(acc.at[pl.ds(s * _RPT, _RPT), :],
                    out.at[c, pl.ds(s * _RPT, _RPT), :])


@functools.lru_cache(maxsize=None)
def _sc_kernels():
    mesh = plsc.VectorSubcoreMesh(core_axis_name="c", subcore_axis_name="s",
                                  num_cores=_NC, num_subcores=_NS)
    params = pltpu.CompilerParams(needs_layout_passes=False)
    sc_dsq = pl.kernel(
        _sc_dsq_body,
        out_type=jax.ShapeDtypeStruct((_E,), jnp.float32),
        mesh=mesh,
        compiler_params=params,
        scratch_types=[
            pltpu.VMEM((_N,), jnp.float32),
            pltpu.VMEM((_N,), jnp.float32),
            pltpu.VMEM((_N,), jnp.float32),
            pltpu.VMEM((_EPW,), jnp.int32),
            pltpu.VMEM((_EPW,), jnp.int32),
            pltpu.VMEM((_EPW,), jnp.float32),
        ],
    )
    sc_conv = pl.kernel(
        _sc_conv_body,
        out_type=jax.ShapeDtypeStruct((_NC, _NP, _D), jnp.float32),
        mesh=mesh,
        compiler_params=params,
        scratch_types=(
            [pltpu.VMEM_SHARED((_NP, _D), jnp.float32)]
            + [pltpu.VMEM((_KB, _D), jnp.float32) for _ in range(2)]
            + [pltpu.VMEM((_KB * (_D // 2),), jnp.int32) for _ in range(2)]
            + [pltpu.VMEM((_KB,), jnp.int32) for _ in range(8)]
            + [pltpu.SemaphoreType.DMA for _ in range(10)]
        ),
    )
    return sc_dsq, sc_conv


def _mm_body(x_ref, w_ref, o_ref):
    o_ref[...] = jnp.dot(x_ref[...], w_ref[...],
                         preferred_element_type=jnp.float32)


def _mm(x, w):
    return pl.pallas_call(
        _mm_body,
        grid=(_NG,),
        in_specs=[pl.BlockSpec((_RB, _D), lambda i: (i, 0)),
                  pl.BlockSpec((_D, _D), lambda i: (0, 0))],
        out_specs=pl.BlockSpec((_RB, _D), lambda i: (i, 0)),
        out_shape=jax.ShapeDtypeStruct((_N, _D), jnp.float32),
    )(x, w)


def _rw_body(dsq_ref, wm1_ref, bm1_ref, wm2_ref, bm2_ref, o_ref):
    d = jnp.sqrt(dsq_ref[...] + 1e-8)
    h = jnp.maximum(d * wm1_ref[...] + bm1_ref[...], 0.0)
    o_ref[...] = (jnp.dot(h, wm2_ref[...],
                          preferred_element_type=jnp.float32)
                  + bm2_ref[...]).astype(jnp.bfloat16)


def _rw_prep(dsq2, wm1, bm1, wm2, bm2):
    return pl.pallas_call(
        _rw_body,
        grid=(_E2 // _EB,),
        in_specs=[pl.BlockSpec((_EB, 1), lambda i: (i, 0)),
                  pl.BlockSpec((1, _H), lambda i: (0, 0)),
                  pl.BlockSpec((1, _H), lambda i: (0, 0)),
                  pl.BlockSpec((_H, _D), lambda i: (0, 0)),
                  pl.BlockSpec((1, _D), lambda i: (0, 0))],
        out_specs=pl.BlockSpec((_EB, _D), lambda i: (i, 0)),
        out_shape=jax.ShapeDtypeStruct((_E2, _D), jnp.bfloat16),
    )(dsq2, wm1, bm1, wm2, bm2)


def _post_mm_body(p_ref, w_ref, f_ref, g_ref):
    x = jnp.maximum(p_ref[0] + p_ref[1], 0.0)
    mu = jnp.mean(x, axis=-1, keepdims=True)
    var = jnp.mean((x - mu) * (x - mu), axis=-1, keepdims=True)
    f = (x - mu) / jnp.sqrt(var + 1e-5)
    f_ref[...] = f
    g_ref[...] = jnp.dot(f, w_ref[...], preferred_element_type=jnp.float32)


def _post_mm_skip_body(p_ref, s_ref, w_ref, f_ref, g_ref):
    x = jnp.maximum(p_ref[0] + p_ref[1], 0.0)
    mu = jnp.mean(x, axis=-1, keepdims=True)
    var = jnp.mean((x - mu) * (x - mu), axis=-1, keepdims=True)
    f = (x - mu) / jnp.sqrt(var + 1e-5) + s_ref[...]
    f_ref[...] = f
    g_ref[...] = jnp.dot(f, w_ref[...], preferred_element_type=jnp.float32)


def _post_plain_body(p_ref, o_ref):
    o_ref[...] = jnp.maximum(p_ref[0] + p_ref[1], 0.0)


_P_SPEC = pl.BlockSpec((_NC, _RB, _D), lambda i: (0, i, 0))
_F_SPEC = pl.BlockSpec((_RB, _D), lambda i: (i, 0))
_W_SPEC = pl.BlockSpec((_D, _D), lambda i: (0, 0))
_F_SHAPE = jax.ShapeDtypeStruct((_N, _D), jnp.float32)


def _post_mm(part, wnext, skipf=None):
    if skipf is not None:
        return pl.pallas_call(
            _post_mm_skip_body, grid=(_NG,),
            in_specs=[_P_SPEC, _F_SPEC, _W_SPEC],
            out_specs=(_F_SPEC, _F_SPEC),
            out_shape=(_F_SHAPE, _F_SHAPE))(part, skipf, wnext)
    return pl.pallas_call(
        _post_mm_body, grid=(_NG,),
        in_specs=[_P_SPEC, _W_SPEC],
        out_specs=(_F_SPEC, _F_SPEC),
        out_shape=(_F_SHAPE, _F_SHAPE))(part, wnext)


def _post_plain(part):
    return pl.pallas_call(
        _post_plain_body, grid=(_NG,),
        in_specs=[_P_SPEC], out_specs=_F_SPEC,
        out_shape=_F_SHAPE)(part)


def kernel(feat, pos, W0_lin, W0_m1, b0_m1, W0_m2, b0_m2, Ws_lin, Ws_m1,
           bs_m1, Ws_m2, bs_m2, W1_lin, W1_m1, b1_m1, W1_m2, b1_m2,
           edge_index):
    src = edge_index[0]
    dst = edge_index[1]
    pos_t = pos.T

    _sc_dsq, _sc_conv = _sc_kernels()
    dsq = _sc_dsq(pos_t[0], pos_t[1], pos_t[2], src, dst)
    dsq2 = jnp.pad(dsq, (0, _E2 - _E)).reshape(_E2, 1)

    perm = jnp.arange(_D).reshape(_D // 32, 2, 16).transpose(0, 2, 1)
    perm = perm.reshape(-1)

    def rwprep(wm1, bm1, wm2, bm2):
        r = _rw_prep(dsq2, wm1, bm1.reshape(1, _H), wm2[:, perm],
                     bm2[perm].reshape(1, _D))
        return lax.bitcast_convert_type(
            r.reshape(_E2 * (_D // 2), 2), jnp.int32)

    rw0 = rwprep(W0_m1, b0_m1, W0_m2, b0_m2)
    rws = rwprep(Ws_m1, bs_m1, Ws_m2, bs_m2)
    rw1 = rwprep(W1_m1, b1_m1, W1_m2, b1_m2)

    g = _mm(feat, W0_lin)
    part = _sc_conv(g, rw0, src, dst)
    f, g = _post_mm(part, Ws_lin)
    for _k in range(2):
        for _i in range(3):
            wnext = Ws_lin if (_i < 2 or _k == 0) else W1_lin
            part = _sc_conv(g, rws, src, dst)
            f, g = _post_mm(part, wnext, skipf=f)
    part = _sc_conv(g, rw1, src, dst)
    return _post_plain(part)


# speedup vs baseline: 24.7129x; 24.7129x over previous
import functools

import jax
import jax.numpy as jnp
from jax import lax
from jax.experimental import pallas as pl
from jax.experimental.pallas import tpu as pltpu
from jax.experimental.pallas import tpu_sc as plsc

_N = 10000
_E = 320000
_D = 128
_H = 32
_NC = 2
_NS = 16
_NW = _NC * _NS
_EPW = _E // _NW
_KB = 80
_NBB = _EPW // _KB
_NP = 10240
_RPT = _NP // _NS
_ZR = 40
_EB = 2048
_E2 = 327680
_RB = 1000
_NG = _N // _RB


def _sc_dsq_body(pos_x, pos_y, pos_z, src, dst, dsq, px, py, pz, sv, dv, ov):
    c = lax.axis_index("c")
    s = lax.axis_index("s")
    wid = c * _NS + s
    base = wid * _EPW
    pltpu.sync_copy(pos_x, px)
    pltpu.sync_copy(pos_y, py)
    pltpu.sync_copy(pos_z, pz)
    pltpu.sync_copy(src.at[pl.ds(base, _EPW)], sv)
    pltpu.sync_copy(dst.at[pl.ds(base, _EPW)], dv)

    def body(i, carry):
        sl = pl.ds(i * 16, 16)
        si = sv[sl]
        di = dv[sl]
        dx = plsc.load_gather(px, [si]) - plsc.load_gather(px, [di])
        dy = plsc.load_gather(py, [si]) - plsc.load_gather(py, [di])
        dz = plsc.load_gather(pz, [si]) - plsc.load_gather(pz, [di])
        ov[sl] = dx * dx + dy * dy + dz * dz
        return carry

    lax.fori_loop(0, _EPW // 16, body, 0)
    pltpu.sync_copy(ov, dsq.at[pl.ds(base, _EPW)])


def _sc_conv_body(g, rw, src, dst, out, acc,
                  rows0, rows1, rwv0, rwv1,
                  si0, si1, si2, si3, di0, di1, di2, di3,
                  gs0, gs1, rs0, rs1, ss0, ss1,
                  is0, is1, is2, is3):
    c = lax.axis_index("c")
    s = lax.axis_index("s")
    wid = c * _NS + s

    rowsb = (rows0, rows1)
    rwvb = (rwv0, rwv1)
    sib = (si0, si1, si2, si3)
    dib = (di0, di1, di2, di3)
    gsem = (gs0, gs1)
    rsem = (rs0, rs1)
    ssem = (ss0, ss1)
    isem = (is0, is1, is2, is3)

    ebase = wid * _EPW
    rwbase = ebase * (_D // 2)

    def zfill(i, carry):
        for j in range(_D // 16):
            z = jnp.zeros((16,), jnp.float32)
            rows0[i, pl.ds(j * 16, 16)] = z
            rows1[i, pl.ds(j * 16, 16)] = z
        return carry

    lax.fori_loop(0, _KB, zfill, 0)
    for r in range(_RPT // _ZR):
        pltpu.sync_copy(rows0.at[pl.ds(0, _ZR), :],
                        acc.at[pl.ds(s * _RPT + r * _ZR, _ZR), :])
    plsc.subcore_barrier()

    def issue_idx(b, r):
        pltpu.async_copy(src.at[pl.ds(ebase + b * _KB, _KB)], sib[r],
                         isem[r])
        pltpu.async_copy(dst.at[pl.ds(ebase + b * _KB, _KB)], dib[r],
                         isem[r])

    def wait_idx(r):
        pltpu.make_async_copy(src.at[pl.ds(0, _KB)], sib[r],
                              isem[r]).wait()
        pltpu.make_async_copy(dst.at[pl.ds(0, _KB)], dib[r],
                              isem[r]).wait()

    _KW = _KB * (_D // 2)

    def issue_gr(b, q, r):
        pltpu.async_copy(g.at[sib[r]], rowsb[q], gsem[q])
        pltpu.async_copy(rw.at[pl.ds(rwbase + b * _KW, _KW)],
                         rwvb[q], rsem[q])

    def wait_gr(q, r):
        pltpu.make_async_copy(g.at[sib[r]], rowsb[q], gsem[q]).wait()
        pltpu.make_async_copy(rw.at[pl.ds(0, _KW)], rwvb[q],
                              rsem[q]).wait()

    def scat(q, r):
        pltpu.async_copy(rowsb[q], acc.at[dib[r]], ssem[q], add=True)

    def wait_s(q):
        pltpu.make_async_copy(rowsb[q], acc.at[dib[0]], ssem[q]).wait()

    def mul(q):
        rows = rowsb[q]
        rwv = rwvb[q]

        @plsc.parallel_loop(0, _KB // 2, unroll=2)
        def _(e2):
            for j in range(_D // 16):
                w = rwv[pl.ds(e2 * _D + j * 16, 16)]
                lo = plsc.bitcast(w << 16, jnp.float32)
                hi = plsc.bitcast(w & jnp.int32(-65536), jnp.float32)
                sl = pl.ds(j * 16, 16)
                rows[2 * e2, sl] = rows[2 * e2, sl] * lo
                rows[2 * e2 + 1, sl] = rows[2 * e2 + 1, sl] * hi

    issue_idx(0, 0)
    issue_idx(1, 1)
    wait_idx(0)
    pltpu.async_copy(rows1, acc.at[dib[0]], ssem[1], add=True)
    issue_gr(0, 0, 0)

    def step(b, q, r):
        qo = 1 - q
        rn1 = (r + 1) % 4
        rn2 = (r + 2) % 4
        wait_s(qo)
        issue_idx(jnp.minimum(b + 2, _NBB - 1), rn2)
        wait_idx(rn1)
        issue_gr(b + 1, qo, rn1)
        wait_gr(q, r)
        mul(q)
        scat(q, r)

    def quad(t, carry):
        b0 = 4 * t
        step(b0, 0, 0)
        step(b0 + 1, 1, 1)
        step(b0 + 2, 0, 2)
        step(b0 + 3, 1, 3)
        return carry

    lax.fori_loop(0, (_NBB - 1) // 4, quad, 0)
    wait_gr(0, 0)
    mul(0)
    scat(0, 0)
    wait_s(1)
    wait_s(0)
    wait_idx(1)
    plsc.subcore_barrier()
    pltpu.sync_copy(acc.at[pl.ds(s * _RPT, _RPT), :],
                    out.at[c, pl.ds(s * _RPT, _RPT), :])


@functools.lru_cache(maxsize=None)
def _sc_kernels():
    mesh = plsc.VectorSubcoreMesh(core_axis_name="c", subcore_axis_name="s",
                                  num_cores=_NC, num_subcores=_NS)
    params = pltpu.CompilerParams(needs_layout_passes=False)
    sc_dsq = pl.kernel(
        _sc_dsq_body,
        out_type=jax.ShapeDtypeStruct((_E,), jnp.float32),
        mesh=mesh,
        compiler_params=params,
        scratch_types=[
            pltpu.VMEM((_N,), jnp.float32),
            pltpu.VMEM((_N,), jnp.float32),
            pltpu.VMEM((_N,), jnp.float32),
            pltpu.VMEM((_EPW,), jnp.int32),
            pltpu.VMEM((_EPW,), jnp.int32),
            pltpu.VMEM((_EPW,), jnp.float32),
        ],
    )
    sc_conv = pl.kernel(
        _sc_conv_body,
        out_type=jax.ShapeDtypeStruct((_NC, _NP, _D), jnp.float32),
        mesh=mesh,
        compiler_params=params,
        scratch_types=(
            [pltpu.VMEM_SHARED((_NP, _D), jnp.float32)]
            + [pltpu.VMEM((_KB, _D), jnp.float32) for _ in range(2)]
            + [pltpu.VMEM((_KB * (_D // 2),), jnp.int32) for _ in range(2)]
            + [pltpu.VMEM((_KB,), jnp.int32) for _ in range(8)]
            + [pltpu.SemaphoreType.DMA for _ in range(10)]
        ),
    )
    return sc_dsq, sc_conv


def _mm_body(x_ref, w_ref, o_ref):
    o_ref[...] = jnp.dot(x_ref[...], w_ref[...],
                         preferred_element_type=jnp.float32)


def _mm(x, w):
    return pl.pallas_call(
        _mm_body,
        grid=(_NG,),
        in_specs=[pl.BlockSpec((_RB, _D), lambda i: (i, 0)),
                  pl.BlockSpec((_D, _D), lambda i: (0, 0))],
        out_specs=pl.BlockSpec((_RB, _D), lambda i: (i, 0)),
        out_shape=jax.ShapeDtypeStruct((_N, _D), jnp.float32),
    )(x, w)


def _rw_body(dsq_ref, wm1_ref, bm1_ref, wm2_ref, bm2_ref, o_ref):
    d = jnp.sqrt(dsq_ref[...] + 1e-8)
    h = jnp.maximum(d * wm1_ref[...] + bm1_ref[...], 0.0)
    v = (jnp.dot(h, wm2_ref[...], preferred_element_type=jnp.float32)
         + bm2_ref[...]).astype(jnp.bfloat16)
    o_ref[...] = pltpu.bitcast(v, jnp.int32)


def _rw_prep(dsq2, wm1, bm1, wm2, bm2):
    return pl.pallas_call(
        _rw_body,
        grid=(_E2 // _EB,),
        in_specs=[pl.BlockSpec((_EB, 1), lambda i: (i, 0)),
                  pl.BlockSpec((1, _H), lambda i: (0, 0)),
                  pl.BlockSpec((1, _H), lambda i: (0, 0)),
                  pl.BlockSpec((_H, _D), lambda i: (0, 0)),
                  pl.BlockSpec((1, _D), lambda i: (0, 0))],
        out_specs=pl.BlockSpec((_EB // 2, _D), lambda i: (i, 0)),
        out_shape=jax.ShapeDtypeStruct((_E2 // 2, _D), jnp.int32),
    )(dsq2, wm1, bm1, wm2, bm2)


def _post_mm_body(p_ref, w_ref, f_ref, g_ref):
    x = jnp.maximum(p_ref[0] + p_ref[1], 0.0)
    mu = jnp.mean(x, axis=-1, keepdims=True)
    var = jnp.mean((x - mu) * (x - mu), axis=-1, keepdims=True)
    f = (x - mu) / jnp.sqrt(var + 1e-5)
    f_ref[...] = f
    g_ref[...] = jnp.dot(f, w_ref[...], preferred_element_type=jnp.float32)


def _post_mm_skip_body(p_ref, s_ref, w_ref, f_ref, g_ref):
    x = jnp.maximum(p_ref[0] + p_ref[1], 0.0)
    mu = jnp.mean(x, axis=-1, keepdims=True)
    var = jnp.mean((x - mu) * (x - mu), axis=-1, keepdims=True)
    f = (x - mu) / jnp.sqrt(var + 1e-5) + s_ref[...]
    f_ref[...] = f
    g_ref[...] = jnp.dot(f, w_ref[...], preferred_element_type=jnp.float32)


def _post_plain_body(p_ref, o_ref):
    o_ref[...] = jnp.maximum(p_ref[0] + p_ref[1], 0.0)


_P_SPEC = pl.BlockSpec((_NC, _RB, _D), lambda i: (0, i, 0))
_F_SPEC = pl.BlockSpec((_RB, _D), lambda i: (i, 0))
_W_SPEC = pl.BlockSpec((_D, _D), lambda i: (0, 0))
_F_SHAPE = jax.ShapeDtypeStruct((_N, _D), jnp.float32)


def _post_mm(part, wnext, skipf=None):
    if skipf is not None:
        return pl.pallas_call(
            _post_mm_skip_body, grid=(_NG,),
            in_specs=[_P_SPEC, _F_SPEC, _W_SPEC],
            out_specs=(_F_SPEC, _F_SPEC),
            out_shape=(_F_SHAPE, _F_SHAPE))(part, skipf, wnext)
    return pl.pallas_call(
        _post_mm_body, grid=(_NG,),
        in_specs=[_P_SPEC, _W_SPEC],
        out_specs=(_F_SPEC, _F_SPEC),
        out_shape=(_F_SHAPE, _F_SHAPE))(part, wnext)


def _post_plain(part):
    return pl.pallas_call(
        _post_plain_body, grid=(_NG,),
        in_specs=[_P_SPEC], out_specs=_F_SPEC,
        out_shape=_F_SHAPE)(part)


def kernel(feat, pos, W0_lin, W0_m1, b0_m1, W0_m2, b0_m2, Ws_lin, Ws_m1,
           bs_m1, Ws_m2, bs_m2, W1_lin, W1_m1, b1_m1, W1_m2, b1_m2,
           edge_index):
    src = edge_index[0]
    dst = edge_index[1]
    pos_t = pos.T

    _sc_dsq, _sc_conv = _sc_kernels()
    dsq = _sc_dsq(pos_t[0], pos_t[1], pos_t[2], src, dst)
    dsq2 = jnp.pad(dsq, (0, _E2 - _E)).reshape(_E2, 1)

    def rwprep(wm1, bm1, wm2, bm2):
        return _rw_prep(dsq2, wm1, bm1.reshape(1, _H), wm2,
                        bm2.reshape(1, _D)).reshape(-1)

    rw0 = rwprep(W0_m1, b0_m1, W0_m2, b0_m2)
    rws = rwprep(Ws_m1, bs_m1, Ws_m2, bs_m2)
    rw1 = rwprep(W1_m1, b1_m1, W1_m2, b1_m2)

    g = _mm(feat, W0_lin)
    part = _sc_conv(g, rw0, src, dst)
    f, g = _post_mm(part, Ws_lin)
    for _k in range(2):
        for _i in range(3):
            wnext = Ws_lin if (_i < 2 or _k == 0) else W1_lin
            part = _sc_conv(g, rws, src, dst)
            f, g = _post_mm(part, wnext, skipf=f)
    part = _sc_conv(g, rw1, src, dst)
    return _post_plain(part)
